# Initial kernel scaffold; baseline (speedup 1.0000x reference)
import jax
import jax.numpy as jnp
from jax.experimental import pallas as pl


def kernel(x, edge_index, W1, b1, W2, b2):
    raise NotImplementedError("write your pallas kernel here")


# speedup vs baseline: 10.6253x; 10.6253x over previous
import functools

import jax
import jax.numpy as jnp
from jax import lax
from jax.experimental import pallas as pl
from jax.experimental.pallas import tpu as pltpu
from jax.experimental.pallas import tpu_sc as plsc

N = 10000
E = 320000
D_IN = 128
D_HID = 128
D_OUT = 64

NC = 2
NS = 16
NW = NC * NS

BATCH = 128
EP = 10112
NB = EP // BATCH
E_PAD = EP * NW
N_PAD = 10240
RPT = N_PAD // NS


def _deg_kernel():
    mesh = plsc.VectorSubcoreMesh(core_axis_name="c", subcore_axis_name="s")

    @functools.partial(
        pl.kernel,
        out_type=jax.ShapeDtypeStruct((NC, N_PAD, 16), jnp.float32),
        mesh=mesh,
        scratch_types=[
            pltpu.VMEM((BATCH,), jnp.int32),
            pltpu.VMEM((BATCH, 16), jnp.float32),
            pltpu.VMEM((BATCH, 16), jnp.float32),
            pltpu.VMEM_SHARED((N_PAD, 16), jnp.float32),
        ],
        compiler_params=pltpu.CompilerParams(use_tc_tiling_on_sc=False),
    )
    def deg(dst_hbm, ones_hbm, out_hbm, dst_v, ones_v, z_v, acc_sh):
        c = lax.axis_index("c")
        s = lax.axis_index("s")
        wid = s * NC + c
        pltpu.sync_copy(ones_hbm, ones_v)
        def zrow(i, _):
            z_v[i, :] = jnp.zeros((16,), jnp.float32)
            return 0
        lax.fori_loop(0, BATCH, zrow, 0)
        for r in range(RPT // BATCH):
            pltpu.sync_copy(z_v, acc_sh.at[pl.ds(s * RPT + r * BATCH, BATCH)])
        plsc.subcore_barrier()

        def body(i, _):
            base = wid * EP + i * BATCH
            pltpu.sync_copy(dst_hbm.at[pl.ds(base, BATCH)], dst_v)
            pltpu.sync_copy(ones_v, acc_sh.at[dst_v], add=True)
            return 0

        lax.fori_loop(0, NB, body, 0)
        plsc.subcore_barrier()
        pltpu.sync_copy(acc_sh.at[pl.ds(s * RPT, RPT)],
                        out_hbm.at[c, pl.ds(s * RPT, RPT)])

    return deg


def _agg_kernel(D):
    mesh = plsc.VectorSubcoreMesh(core_axis_name="c", subcore_axis_name="s")

    @functools.partial(
        pl.kernel,
        out_type=jax.ShapeDtypeStruct((NC, N_PAD, D), jnp.float32),
        mesh=mesh,
        scratch_types=[
            pltpu.VMEM((BATCH,), jnp.int32),
            pltpu.VMEM((BATCH,), jnp.int32),
            pltpu.VMEM((BATCH, D), jnp.float32),
            pltpu.VMEM_SHARED((N_PAD, D), jnp.float32),
            pltpu.SemaphoreType.DMA,
        ],
        compiler_params=pltpu.CompilerParams(use_tc_tiling_on_sc=False),
    )
    def agg(table_hbm, src_hbm, dst_hbm, out_hbm,
            src_v, dst_v, rows_v, acc_sh, sem):
        c = lax.axis_index("c")
        s = lax.axis_index("s")
        wid = s * NC + c

        def zrow(i, _):
            for j in range(D // 16):
                rows_v[i, pl.ds(j * 16, 16)] = jnp.zeros((16,), jnp.float32)
            return 0
        lax.fori_loop(0, BATCH, zrow, 0)
        for r in range(RPT // BATCH):
            pltpu.sync_copy(rows_v, acc_sh.at[pl.ds(s * RPT + r * BATCH, BATCH)])
        plsc.subcore_barrier()

        def body(i, _):
            base = wid * EP + i * BATCH
            pltpu.sync_copy(src_hbm.at[pl.ds(base, BATCH)], src_v)
            pltpu.sync_copy(dst_hbm.at[pl.ds(base, BATCH)], dst_v)
            pltpu.async_copy(table_hbm.at[src_v], rows_v, sem).wait()
            pltpu.sync_copy(rows_v, acc_sh.at[dst_v], add=True)
            return 0

        lax.fori_loop(0, NB, body, 0)
        plsc.subcore_barrier()
        pltpu.sync_copy(acc_sh.at[pl.ds(s * RPT, RPT)],
                        out_hbm.at[c, pl.ds(s * RPT, RPT)])

    return agg


_ROWS_BLK = 1000
_GRID = N // _ROWS_BLK


def _dis_from(degp_blk):
    deg = degp_blk[0, :, :1] + degp_blk[1, :, :1] + 1.0
    return lax.rsqrt(deg)


def _tc1_body(degp_ref, x_ref, w1_ref, g1_ref):
    dis = _dis_from(degp_ref[...])
    h = jnp.dot(x_ref[...], w1_ref[...], preferred_element_type=jnp.float32)
    g1_ref[...] = dis * h


def _tc2_body(degp_ref, r1_ref, g1_ref, b1_ref, w2_ref, g2_ref):
    dis = _dis_from(degp_ref[...])
    a = dis * (r1_ref[0] + r1_ref[1] + g1_ref[...]) + b1_ref[...]
    x2 = jnp.maximum(a, 0.0)
    g2_ref[...] = dis * jnp.dot(x2, w2_ref[...],
                                preferred_element_type=jnp.float32)


def _tc3_body(degp_ref, r2_ref, g2_ref, b2_ref, out_ref):
    dis = _dis_from(degp_ref[...])
    out_ref[...] = dis * (r2_ref[0] + r2_ref[1] + g2_ref[...]) + b2_ref[...]


def _blk_parts(d):
    return pl.BlockSpec((NC, _ROWS_BLK, d), lambda i: (0, i, 0))


def _blk_rows(d):
    return pl.BlockSpec((_ROWS_BLK, d), lambda i: (i, 0))


def _blk_full(shape):
    return pl.BlockSpec(shape, lambda i: tuple(0 for _ in shape))


def kernel(x, edge_index, W1, b1, W2, b2):
    src = edge_index[0]
    dst = edge_index[1]
    pad = E_PAD - E
    src_p = jnp.concatenate([src, jnp.zeros((pad,), jnp.int32)])
    dst_p = jnp.concatenate([dst, jnp.full((pad,), N, jnp.int32)])
    ones16 = jnp.ones((BATCH, 16), jnp.float32)

    degp = _deg_kernel()(dst_p, ones16)

    g1 = pl.pallas_call(
        _tc1_body,
        grid=(_GRID,),
        in_specs=[_blk_parts(16), _blk_rows(D_IN), _blk_full((D_IN, D_HID))],
        out_specs=_blk_rows(D_HID),
        out_shape=jax.ShapeDtypeStruct((N, D_HID), jnp.float32),
    )(degp, x, W1)

    r1 = _agg_kernel(D_HID)(g1, src_p, dst_p)

    g2 = pl.pallas_call(
        _tc2_body,
        grid=(_GRID,),
        in_specs=[_blk_parts(16), _blk_parts(D_HID), _blk_rows(D_HID),
                  _blk_full((1, D_HID)), _blk_full((D_HID, D_OUT))],
        out_specs=_blk_rows(D_OUT),
        out_shape=jax.ShapeDtypeStruct((N, D_OUT), jnp.float32),
    )(degp, r1, g1, b1.reshape(1, D_HID), W2)

    r2 = _agg_kernel(D_OUT)(g2, src_p, dst_p)

    out = pl.pallas_call(
        _tc3_body,
        grid=(_GRID,),
        in_specs=[_blk_parts(16), _blk_parts(D_OUT), _blk_rows(D_OUT),
                  _blk_full((1, D_OUT))],
        out_specs=_blk_rows(D_OUT),
        out_shape=jax.ShapeDtypeStruct((N, D_OUT), jnp.float32),
    )(degp, r2, g2, b2.reshape(1, D_OUT))

    return out
